# baseline (speedup 1.0000x reference)
import functools

import jax
import jax.numpy as jnp
from jax import lax
from jax.experimental import pallas as pl
from jax.experimental.pallas import tpu as pltpu
from jax.experimental.pallas import tpu_sc as plsc


def _proj_body(table_ref, w_ref, b_ref, out_ref):
    out_ref[...] = (
        jnp.dot(table_ref[...], w_ref[...], preferred_element_type=jnp.float32)
        + b_ref[...]
    )


def _project_table(table, W, b):
    V, _ = table.shape
    N = W.shape[1]
    return pl.pallas_call(
        _proj_body,
        out_shape=jax.ShapeDtypeStruct((V, N), jnp.float32),
    )(table, W, b.reshape(1, N))


def _make_sc_gather(Bc, Lp, D, V, n_workers, chunk_b, idx_row):
    b_per_w = Bc // n_workers
    n_chunks = b_per_w // chunk_b
    mesh = plsc.VectorSubcoreMesh(core_axis_name="c", subcore_axis_name="s")
    num_cores = 2

    @functools.partial(
        pl.kernel,
        mesh=mesh,
        out_type=jax.ShapeDtypeStruct((Bc * Lp, D), jnp.float32),
        scratch_types=[
            pltpu.VMEM((n_chunks, idx_row), jnp.int32),
            pltpu.VMEM((2, idx_row, D), jnp.float32),
            pltpu.VMEM_SHARED((V, D), jnp.float32),
            pltpu.SemaphoreType.DMA,
            pltpu.SemaphoreType.DMA,
            pltpu.SemaphoreType.DMA,
        ],
    )
    def sc_gather(idx_hbm, proj_hbm, out_hbm, idx_v, rows_v, tab_sh, gsem, osem0, osem1):
        wid = lax.axis_index("s") * num_cores + lax.axis_index("c")
        base_b = wid * b_per_w

        @pl.when(lax.axis_index("s") == 0)
        def _stage_table():
            pltpu.sync_copy(proj_hbm, tab_sh)

        pltpu.sync_copy(idx_hbm.at[wid], idx_v)
        plsc.subcore_barrier()

        osems = (osem0, osem1)

        def gather_chunk(c, buf):
            pltpu.async_copy(tab_sh.at[idx_v.at[c]], rows_v.at[buf], gsem).wait()

        def out_copy(c, buf):
            b0 = base_b + c * chunk_b
            return pltpu.make_async_copy(
                rows_v.at[buf, pl.ds(0, chunk_b * Lp)],
                out_hbm.at[pl.ds(b0 * Lp, chunk_b * Lp)],
                osems[buf],
            )

        gather_chunk(0, 0)
        out_copy(0, 0).start()
        gather_chunk(1, 1)
        out_copy(1, 1).start()

        def body(c0):
            for off in range(2):
                c = c0 + off
                buf = off
                out_copy(c - 2, buf).wait()
                gather_chunk(c, buf)
                out_copy(c, buf).start()

        pl.loop(2, n_chunks, step=2)(body)

        out_copy(n_chunks - 2, 0).wait()
        out_copy(n_chunks - 1, 1).wait()

    return sc_gather


def _make_assemble(B, L, Lp, D, Bc, k, nb, first):
    n_blocks = Bc // nb

    def body(*refs):
        in_ref, out_ref = refs[-2], refs[-1]
        for i in range(nb):
            out_ref[i] = in_ref[pl.ds(i * Lp, L), :]

    in_specs = [pl.BlockSpec((nb * Lp, D), lambda j: (j, 0))]
    aliases = {}
    if not first:
        in_specs = [pl.BlockSpec(memory_space=pl.ANY)] + in_specs
        aliases = {0: 0}
    return pl.pallas_call(
        body,
        grid=(n_blocks,),
        in_specs=in_specs,
        out_specs=pl.BlockSpec(
            (nb, L, D), lambda j, k=k, n=n_blocks: (k * n + j, 0, 0)
        ),
        out_shape=jax.ShapeDtypeStruct((B, L, D), jnp.float32),
        input_output_aliases=aliases,
    )


def kernel(categories, table, W, b):
    B, L = categories.shape
    V, _ = table.shape
    D = W.shape[1]
    Lp = (L + 7) // 8 * 8

    n_workers = 32
    chunk_b = 2
    idx_row = 128
    n_calls = 4
    nb = 8
    Bc = B // n_calls
    assert Bc % (n_workers * chunk_b) == 0 and chunk_b * Lp <= idx_row
    assert Bc % nb == 0

    proj = _project_table(table, W, b)
    idx = jnp.pad(categories.astype(jnp.int32), ((0, 0), (0, Lp - L)))
    idx = idx.reshape(B // chunk_b, chunk_b * Lp)
    idx = jnp.pad(idx, ((0, 0), (0, idx_row - chunk_b * Lp)))
    idx = idx.reshape(n_calls, n_workers, Bc // (n_workers * chunk_b), idx_row)

    gather = _make_sc_gather(Bc, Lp, D, V, n_workers, chunk_b, idx_row)
    chunks = [gather(idx[k], proj) for k in range(n_calls)]

    out = _make_assemble(B, L, Lp, D, Bc, 0, nb, True)(chunks[0])
    for k in range(1, n_calls):
        out = _make_assemble(B, L, Lp, D, Bc, k, nb, False)(out, chunks[k])
    return out


# speedup vs baseline: 3.0157x; 3.0157x over previous
import functools

import jax
import jax.numpy as jnp
from jax import lax
from jax.experimental import pallas as pl
from jax.experimental.pallas import tpu as pltpu
from jax.experimental.pallas import tpu_sc as plsc


def _proj_body(table_ref, w_ref, b_ref, out_ref):
    out_ref[...] = (
        jnp.dot(table_ref[...], w_ref[...], preferred_element_type=jnp.float32)
        + b_ref[...]
    )


def _project_table(table, W, b):
    V, _ = table.shape
    N = W.shape[1]
    return pl.pallas_call(
        _proj_body,
        out_shape=jax.ShapeDtypeStruct((V, N), jnp.float32),
    )(table, W, b.reshape(1, N))


def _make_sc_gather(B, L, D, V, n_workers, chunk_b, idx_row):
    b_per_w = B // n_workers
    n_chunks = b_per_w // chunk_b
    mesh = plsc.VectorSubcoreMesh(core_axis_name="c", subcore_axis_name="s")
    num_cores = 2

    @functools.partial(
        pl.kernel,
        mesh=mesh,
        out_type=jax.ShapeDtypeStruct((B, L, D), jnp.float32),
        compiler_params=pltpu.CompilerParams(use_tc_tiling_on_sc=True),
        scratch_types=[
            pltpu.VMEM((n_chunks, idx_row), jnp.int32),
            pltpu.VMEM((2, idx_row, D), jnp.float32),
            pltpu.VMEM_SHARED((V, D), jnp.float32),
            pltpu.SemaphoreType.DMA,
            pltpu.SemaphoreType.DMA,
            pltpu.SemaphoreType.DMA,
        ],
    )
    def sc_gather(idx_hbm, proj_hbm, out_hbm, idx_v, rows_v, tab_sh, gsem, osem0, osem1):
        wid = lax.axis_index("s") * num_cores + lax.axis_index("c")
        base_b = wid * b_per_w

        @pl.when(lax.axis_index("s") == 0)
        def _stage_table():
            pltpu.sync_copy(proj_hbm, tab_sh)

        pltpu.sync_copy(idx_hbm.at[wid], idx_v)
        plsc.subcore_barrier()

        osems = (osem0, osem1)

        def gather_chunk(c, buf):
            pltpu.async_copy(tab_sh.at[idx_v.at[c]], rows_v.at[buf], gsem).wait()

        def out_copies(c, buf):
            b0 = base_b + c * chunk_b
            return [
                pltpu.make_async_copy(
                    rows_v.at[buf, pl.ds(i * L, L)],
                    out_hbm.at[b0 + i],
                    osems[buf],
                )
                for i in range(chunk_b)
            ]

        def start_out(c, buf):
            for cp in out_copies(c, buf):
                cp.start()

        def wait_out(c_prev, buf):
            for cp in out_copies(c_prev, buf):
                cp.wait()

        gather_chunk(0, 0)
        start_out(0, 0)
        gather_chunk(1, 1)
        start_out(1, 1)

        def body(c0):
            for off in range(2):
                c = c0 + off
                buf = off
                wait_out(c - 2, buf)
                gather_chunk(c, buf)
                start_out(c, buf)

        pl.loop(2, n_chunks, step=2)(body)

        wait_out(n_chunks - 2, 0)
        wait_out(n_chunks - 1, 1)

    return sc_gather


def kernel(categories, table, W, b):
    B, L = categories.shape
    V, _ = table.shape
    D = W.shape[1]

    n_workers = 32
    chunk_b = 2
    idx_row = 128
    assert B % (n_workers * chunk_b) == 0 and chunk_b * L <= idx_row

    proj = _project_table(table, W, b)
    idx = categories.astype(jnp.int32).reshape(B // chunk_b, chunk_b * L)
    idx = jnp.pad(idx, ((0, 0), (0, idx_row - chunk_b * L)))
    idx = idx.reshape(n_workers, B // (n_workers * chunk_b), idx_row)
    return _make_sc_gather(B, L, D, V, n_workers, chunk_b, idx_row)(idx, proj)
